# Initial kernel scaffold; baseline (speedup 1.0000x reference)
import jax
import jax.numpy as jnp
from jax.experimental import pallas as pl


def kernel(node_feat_0, node_feat_1, edge_vec, distances, idx_i, idx_j, atomic_number, W_rbf, W0a, W0b, W0c, W1a, W1b, Wg, Wn, bg, bn, centers):
    raise NotImplementedError("write your pallas kernel here")


# speedup vs baseline: 18.4728x; 18.4728x over previous
import functools

import jax
import jax.numpy as jnp
from jax import lax
from jax.experimental import pallas as pl
from jax.experimental.pallas import tpu as pltpu
from jax.experimental.pallas import tpu_sc as plsc

N = 10000
E = 320000
C = 128
NRBF = 16
GAMMA = 4.0
INV_NORM = 1.0 / 32.0

NC = 2
NS = 16
NW = NC * NS
EPT = E // NW
CHUNK = 80
STEPS = EPT // CHUNK
NPAD = 10240
RPT = NPAD // NS

EB = 1280
NB = 1000


def _mesh():
    return plsc.VectorSubcoreMesh(core_axis_name="c", subcore_axis_name="s",
                                  num_cores=NC, num_subcores=NS)


def _gather(nf0, nf1x, nf1y, nf1z, idxj2):
    @functools.partial(
        pl.kernel,
        out_type=jax.ShapeDtypeStruct((4 * E, C), jnp.float32),
        mesh=_mesh(),
        scratch_types=[
            pltpu.VMEM((STEPS, CHUNK), jnp.int32),
            pltpu.VMEM((4, CHUNK, C), jnp.float32),
            pltpu.SemaphoreType.DMA,
        ],
    )
    def k(nf0_h, nf1x_h, nf1y_h, nf1z_h, idx_h, out_h, idx_v, rows_v, sem):
        c = lax.axis_index("c")
        s = lax.axis_index("s")
        wid = s * NC + c
        pltpu.sync_copy(idx_h.at[wid], idx_v)
        planes = (nf0_h, nf1x_h, nf1y_h, nf1z_h)

        def step(j, carry):
            irow = idx_v.at[j]
            cps = [
                pltpu.async_copy(planes[p].at[irow], rows_v.at[p], sem)
                for p in range(4)
            ]
            for cp in cps:
                cp.wait()
            e0 = wid * EPT + j * CHUNK
            for p in range(4):
                pltpu.sync_copy(rows_v.at[p], out_h.at[pl.ds(p * E + e0, CHUNK)])
            return carry

        lax.fori_loop(0, STEPS, step, 0, unroll=False)

    return k(nf0, nf1x, nf1y, nf1z, idxj2)


def _messages(g, edge_vec, dist2, centers2, w_rbf):
    def body(ev_ref, d_ref, cen_ref, w_ref, g_ref, m_ref):
        ev = ev_ref[...]
        nrm = jnp.sqrt(jnp.sum(ev * ev, axis=1, keepdims=True))
        evn = ev / (nrm + 1e-8)
        d = d_ref[...]
        diff = d - cen_ref[...]
        rbf = jnp.exp(-GAMMA * diff * diff)
        w = w_ref[...]
        f0 = jnp.dot(rbf, w[0], preferred_element_type=jnp.float32)
        f1 = jnp.dot(rbf, w[1], preferred_element_type=jnp.float32)
        f2 = jnp.dot(rbf, w[2], preferred_element_type=jnp.float32)
        f3 = jnp.dot(rbf, w[3], preferred_element_type=jnp.float32)
        g0 = g_ref[0]
        g1x = g_ref[1]
        g1y = g_ref[2]
        g1z = g_ref[3]
        evx = evn[:, 0:1]
        evy = evn[:, 1:2]
        evz = evn[:, 2:3]
        dot1 = g1x * evx + g1y * evy + g1z * evz
        m_ref[0] = f0 * g0 + f3 * dot1
        t = f1 * g0
        m_ref[1] = t * evx + f2 * g1x
        m_ref[2] = t * evy + f2 * g1y
        m_ref[3] = t * evz + f2 * g1z

    return pl.pallas_call(
        body,
        grid=(E // EB,),
        in_specs=[
            pl.BlockSpec((EB, 3), lambda i: (i, 0)),
            pl.BlockSpec((EB, 1), lambda i: (i, 0)),
            pl.BlockSpec((1, NRBF), lambda i: (0, 0)),
            pl.BlockSpec((4, NRBF, C), lambda i: (0, 0, 0)),
            pl.BlockSpec((4, EB, C), lambda i: (0, i, 0)),
        ],
        out_specs=pl.BlockSpec((4, EB, C), lambda i: (0, i, 0)),
        out_shape=jax.ShapeDtypeStruct((4, E, C), jnp.float32),
    )(edge_vec, dist2, centers2, w_rbf, g)


def _scatter(msgs_flat, idxi2, zeros):
    @functools.partial(
        pl.kernel,
        out_type=jax.ShapeDtypeStruct((4 * NPAD, C), jnp.float32),
        mesh=_mesh(),
        scratch_types=[
            pltpu.VMEM((2 * STEPS, CHUNK), jnp.int32),
            pltpu.VMEM((CHUNK, C), jnp.float32),
            pltpu.VMEM_SHARED((NPAD, C), jnp.float32),
            pltpu.SemaphoreType.DMA,
        ],
    )
    def k(m_h, idx_h, z_h, out_h, idx_v, buf_v, acc, sem):
        c = lax.axis_index("c")
        s = lax.axis_index("s")
        pltpu.sync_copy(idx_h.at[s], idx_v)
        row0 = s * RPT
        for kk in range(2):
            chunk = kk * NC + c
            pltpu.sync_copy(z_h.at[pl.ds(row0, RPT)], acc.at[pl.ds(row0, RPT)])
            plsc.subcore_barrier()

            def step(j, carry):
                e0 = s * (2 * EPT) + j * CHUNK
                pltpu.sync_copy(m_h.at[pl.ds(chunk * E + e0, CHUNK)], buf_v)
                pltpu.sync_copy(buf_v, acc.at[idx_v.at[j]], add=True)
                return carry

            lax.fori_loop(0, 2 * STEPS, step, 0, unroll=False)
            plsc.subcore_barrier()
            pltpu.sync_copy(
                acc.at[pl.ds(row0, RPT)],
                out_h.at[pl.ds(chunk * NPAD + row0, RPT)],
            )

    return k(msgs_flat, idxi2, zeros)


def _node(rcat, nf0, nf1x, nf1y, nf1z, w0a, w0b, w0c, w1a, w1b, wg, wn, bg2, bn2):
    def body(r_ref, n0_ref, x_ref, y_ref, z_ref, a_ref, b_ref, c_ref, d_ref,
             e_ref, g_ref, n_ref, bg_ref, bn_ref, o0_ref, o1_ref):
        r0 = r_ref[0] * INV_NORM
        r1x = r_ref[1] * INV_NORM
        r1y = r_ref[2] * INV_NORM
        r1z = r_ref[3] * INV_NORM
        W0a = a_ref[...]
        W0b = b_ref[...]
        W0c = c_ref[...]
        W1a = d_ref[...]
        W1b = e_ref[...]
        Wg = g_ref[...]
        Wn = n_ref[...]
        y0 = (jnp.dot(r0, W0a, preferred_element_type=jnp.float32)
              + jnp.dot(r0 * r0, W0b, preferred_element_type=jnp.float32)
              + jnp.dot(r1x * r1x + r1y * r1y + r1z * r1z, W0c,
                        preferred_element_type=jnp.float32))
        gate = jax.nn.silu(jnp.dot(y0, Wg, preferred_element_type=jnp.float32)
                           + bg_ref[...])
        u0 = jax.nn.silu(jnp.dot(y0, Wn, preferred_element_type=jnp.float32)
                         + bn_ref[...])
        o0_ref[...] = n0_ref[...] + u0
        y1x = (jnp.dot(r1x, W1a, preferred_element_type=jnp.float32)
               + jnp.dot(r0 * r1x, W1b, preferred_element_type=jnp.float32))
        y1y = (jnp.dot(r1y, W1a, preferred_element_type=jnp.float32)
               + jnp.dot(r0 * r1y, W1b, preferred_element_type=jnp.float32))
        y1z = (jnp.dot(r1z, W1a, preferred_element_type=jnp.float32)
               + jnp.dot(r0 * r1z, W1b, preferred_element_type=jnp.float32))
        o1_ref[0] = x_ref[...] + y1x * gate
        o1_ref[1] = y_ref[...] + y1y * gate
        o1_ref[2] = z_ref[...] + y1z * gate

    return pl.pallas_call(
        body,
        grid=(N // NB,),
        in_specs=[
            pl.BlockSpec((4, NB, C), lambda i: (0, i, 0)),
            pl.BlockSpec((NB, C), lambda i: (i, 0)),
            pl.BlockSpec((NB, C), lambda i: (i, 0)),
            pl.BlockSpec((NB, C), lambda i: (i, 0)),
            pl.BlockSpec((NB, C), lambda i: (i, 0)),
            pl.BlockSpec((C, C), lambda i: (0, 0)),
            pl.BlockSpec((C, C), lambda i: (0, 0)),
            pl.BlockSpec((C, C), lambda i: (0, 0)),
            pl.BlockSpec((C, C), lambda i: (0, 0)),
            pl.BlockSpec((C, C), lambda i: (0, 0)),
            pl.BlockSpec((C, C), lambda i: (0, 0)),
            pl.BlockSpec((C, C), lambda i: (0, 0)),
            pl.BlockSpec((1, C), lambda i: (0, 0)),
            pl.BlockSpec((1, C), lambda i: (0, 0)),
        ],
        out_specs=[
            pl.BlockSpec((NB, C), lambda i: (i, 0)),
            pl.BlockSpec((3, NB, C), lambda i: (0, i, 0)),
        ],
        out_shape=[
            jax.ShapeDtypeStruct((N, C), jnp.float32),
            jax.ShapeDtypeStruct((3, N, C), jnp.float32),
        ],
    )(rcat, nf0, nf1x, nf1y, nf1z, w0a, w0b, w0c, w1a, w1b, wg, wn, bg2, bn2)


def kernel(node_feat_0, node_feat_1, edge_vec, distances, idx_i, idx_j,
           atomic_number, W_rbf, W0a, W0b, W0c, W1a, W1b, Wg, Wn, bg, bn,
           centers):
    nf1p = jnp.transpose(node_feat_1, (2, 0, 1))
    nf1x, nf1y, nf1z = nf1p[0], nf1p[1], nf1p[2]
    idxj2 = idx_j.astype(jnp.int32).reshape(NW, STEPS, CHUNK)
    idxi2 = idx_i.astype(jnp.int32).reshape(NS, 2 * STEPS, CHUNK)

    g = _gather(node_feat_0, nf1x, nf1y, nf1z, idxj2).reshape(4, E, C)
    msgs = _messages(g, edge_vec, distances.reshape(E, 1),
                     centers.reshape(1, NRBF), W_rbf)
    zeros = jnp.zeros((NPAD, C), jnp.float32)
    rcat = _scatter(msgs.reshape(4 * E, C), idxi2,
                    zeros).reshape(4, NPAD, C)[:, :N, :]
    out0, out1p = _node(rcat, node_feat_0, nf1x, nf1y, nf1z,
                        W0a, W0b, W0c, W1a, W1b, Wg, Wn,
                        bg.reshape(1, C), bn.reshape(1, C))
    return out0, jnp.transpose(out1p, (1, 2, 0))


# speedup vs baseline: 18.7859x; 1.0169x over previous
import functools

import jax
import jax.numpy as jnp
from jax import lax
from jax.experimental import pallas as pl
from jax.experimental.pallas import tpu as pltpu
from jax.experimental.pallas import tpu_sc as plsc

N = 10000
E = 320000
C = 128
NRBF = 16
GAMMA = 4.0
INV_NORM = 1.0 / 32.0

NC = 2
NS = 16
NW = NC * NS
EPT = E // NW
CHUNK = 80
STEPS = EPT // CHUNK
NPAD = 10240
RPT = NPAD // NS

EB = 1280
NB = 1000


def _mesh():
    return plsc.VectorSubcoreMesh(core_axis_name="c", subcore_axis_name="s",
                                  num_cores=NC, num_subcores=NS)


def _gather(tbl, idxj2):
    @functools.partial(
        pl.kernel,
        out_type=jax.ShapeDtypeStruct((E, 4 * C), jnp.float32),
        mesh=_mesh(),
        scratch_types=[
            pltpu.VMEM((STEPS, CHUNK), jnp.int32),
            pltpu.VMEM((2, CHUNK, 4 * C), jnp.float32),
            pltpu.SemaphoreType.DMA,
            pltpu.SemaphoreType.DMA,
            pltpu.SemaphoreType.DMA,
            pltpu.SemaphoreType.DMA,
        ],
    )
    def k(tbl_h, idx_h, out_h, idx_v, rows_v, g0, g1, w0, w1):
        c = lax.axis_index("c")
        s = lax.axis_index("s")
        wid = s * NC + c
        pltpu.sync_copy(idx_h.at[wid], idx_v)
        gsem = (g0, g1)
        wsem = (w0, w1)

        def gcp(b, j):
            return pltpu.make_async_copy(tbl_h.at[idx_v.at[j]], rows_v.at[b],
                                         gsem[b])

        def wcp(b, j):
            e0 = wid * EPT + j * CHUNK
            return pltpu.make_async_copy(rows_v.at[b],
                                         out_h.at[pl.ds(e0, CHUNK)], wsem[b])

        gcp(0, 0).start()

        def pair(jj, carry):
            j0 = 2 * jj
            j1 = j0 + 1
            gcp(1, j1).start()
            gcp(0, j0).wait()
            wcp(0, j0).start()
            gcp(1, j1).wait()
            wcp(1, j1).start()
            wcp(0, j0).wait()
            gcp(0, j0 + 2).start()
            wcp(1, j1).wait()
            return carry

        lax.fori_loop(0, (STEPS - 1) // 2, pair, 0, unroll=False)
        jt = STEPS - 1
        gcp(0, jt).wait()
        wcp(0, jt).start()
        wcp(0, jt).wait()

    return k(tbl, idxj2)


def _messages(g, edge_vec, dist2, centers2, w_rbf):
    def body(ev_ref, d_ref, cen_ref, w_ref, g_ref, m_ref):
        ev = ev_ref[...]
        nrm = jnp.sqrt(jnp.sum(ev * ev, axis=1, keepdims=True))
        evn = ev / (nrm + 1e-8)
        d = d_ref[...]
        diff = d - cen_ref[...]
        rbf = jnp.exp(-GAMMA * diff * diff)
        w = w_ref[...]
        f0 = jnp.dot(rbf, w[0], preferred_element_type=jnp.float32)
        f1 = jnp.dot(rbf, w[1], preferred_element_type=jnp.float32)
        f2 = jnp.dot(rbf, w[2], preferred_element_type=jnp.float32)
        f3 = jnp.dot(rbf, w[3], preferred_element_type=jnp.float32)
        gall = g_ref[...]
        g0 = gall[:, 0 * C:1 * C]
        g1x = gall[:, 1 * C:2 * C]
        g1y = gall[:, 2 * C:3 * C]
        g1z = gall[:, 3 * C:4 * C]
        evx = evn[:, 0:1]
        evy = evn[:, 1:2]
        evz = evn[:, 2:3]
        dot1 = g1x * evx + g1y * evy + g1z * evz
        m_ref[0] = f0 * g0 + f3 * dot1
        t = f1 * g0
        m_ref[1] = t * evx + f2 * g1x
        m_ref[2] = t * evy + f2 * g1y
        m_ref[3] = t * evz + f2 * g1z

    return pl.pallas_call(
        body,
        grid=(E // EB,),
        in_specs=[
            pl.BlockSpec((EB, 3), lambda i: (i, 0)),
            pl.BlockSpec((EB, 1), lambda i: (i, 0)),
            pl.BlockSpec((1, NRBF), lambda i: (0, 0)),
            pl.BlockSpec((4, NRBF, C), lambda i: (0, 0, 0)),
            pl.BlockSpec((EB, 4 * C), lambda i: (i, 0)),
        ],
        out_specs=pl.BlockSpec((4, EB, C), lambda i: (0, i, 0)),
        out_shape=jax.ShapeDtypeStruct((4, E, C), jnp.float32),
    )(edge_vec, dist2, centers2, w_rbf, g)


def _scatter(msgs_flat, idxi2, zeros):
    @functools.partial(
        pl.kernel,
        out_type=jax.ShapeDtypeStruct((4 * NPAD, C), jnp.float32),
        mesh=_mesh(),
        scratch_types=[
            pltpu.VMEM((2 * STEPS, CHUNK), jnp.int32),
            pltpu.VMEM((CHUNK, C), jnp.float32),
            pltpu.VMEM_SHARED((NPAD, C), jnp.float32),
            pltpu.SemaphoreType.DMA,
        ],
    )
    def k(m_h, idx_h, z_h, out_h, idx_v, buf_v, acc, l0):
        c = lax.axis_index("c")
        s = lax.axis_index("s")
        pltpu.sync_copy(idx_h.at[s], idx_v)
        row0 = s * RPT
        nstep = 2 * STEPS

        for kk in range(2):
            chunk = kk * NC + c

            pltpu.sync_copy(z_h.at[pl.ds(row0, RPT)], acc.at[pl.ds(row0, RPT)])
            plsc.subcore_barrier()

            def step(j, carry):
                e0 = s * (2 * EPT) + j * CHUNK
                pltpu.sync_copy(m_h.at[pl.ds(chunk * E + e0, CHUNK)], buf_v)
                pltpu.sync_copy(buf_v, acc.at[idx_v.at[j]], add=True)
                return carry

            lax.fori_loop(0, nstep, step, 0, unroll=False)

            plsc.subcore_barrier()
            pltpu.sync_copy(
                acc.at[pl.ds(row0, RPT)],
                out_h.at[pl.ds(chunk * NPAD + row0, RPT)],
            )

    return k(msgs_flat, idxi2, zeros)


def _node(rcat, nf0, nf1x, nf1y, nf1z, w0a, w0b, w0c, w1a, w1b, wg, wn, bg2, bn2):
    def body(r_ref, n0_ref, x_ref, y_ref, z_ref, a_ref, b_ref, c_ref, d_ref,
             e_ref, g_ref, n_ref, bg_ref, bn_ref, o0_ref, o1_ref):
        r0 = r_ref[0] * INV_NORM
        r1x = r_ref[1] * INV_NORM
        r1y = r_ref[2] * INV_NORM
        r1z = r_ref[3] * INV_NORM
        W0a = a_ref[...]
        W0b = b_ref[...]
        W0c = c_ref[...]
        W1a = d_ref[...]
        W1b = e_ref[...]
        Wg = g_ref[...]
        Wn = n_ref[...]
        y0 = (jnp.dot(r0, W0a, preferred_element_type=jnp.float32)
              + jnp.dot(r0 * r0, W0b, preferred_element_type=jnp.float32)
              + jnp.dot(r1x * r1x + r1y * r1y + r1z * r1z, W0c,
                        preferred_element_type=jnp.float32))
        gate = jax.nn.silu(jnp.dot(y0, Wg, preferred_element_type=jnp.float32)
                           + bg_ref[...])
        u0 = jax.nn.silu(jnp.dot(y0, Wn, preferred_element_type=jnp.float32)
                         + bn_ref[...])
        o0_ref[...] = n0_ref[...] + u0
        y1x = (jnp.dot(r1x, W1a, preferred_element_type=jnp.float32)
               + jnp.dot(r0 * r1x, W1b, preferred_element_type=jnp.float32))
        y1y = (jnp.dot(r1y, W1a, preferred_element_type=jnp.float32)
               + jnp.dot(r0 * r1y, W1b, preferred_element_type=jnp.float32))
        y1z = (jnp.dot(r1z, W1a, preferred_element_type=jnp.float32)
               + jnp.dot(r0 * r1z, W1b, preferred_element_type=jnp.float32))
        o1_ref[0] = x_ref[...] + y1x * gate
        o1_ref[1] = y_ref[...] + y1y * gate
        o1_ref[2] = z_ref[...] + y1z * gate

    return pl.pallas_call(
        body,
        grid=(N // NB,),
        in_specs=[
            pl.BlockSpec((4, NB, C), lambda i: (0, i, 0)),
            pl.BlockSpec((NB, C), lambda i: (i, 0)),
            pl.BlockSpec((NB, C), lambda i: (i, 0)),
            pl.BlockSpec((NB, C), lambda i: (i, 0)),
            pl.BlockSpec((NB, C), lambda i: (i, 0)),
            pl.BlockSpec((C, C), lambda i: (0, 0)),
            pl.BlockSpec((C, C), lambda i: (0, 0)),
            pl.BlockSpec((C, C), lambda i: (0, 0)),
            pl.BlockSpec((C, C), lambda i: (0, 0)),
            pl.BlockSpec((C, C), lambda i: (0, 0)),
            pl.BlockSpec((C, C), lambda i: (0, 0)),
            pl.BlockSpec((C, C), lambda i: (0, 0)),
            pl.BlockSpec((1, C), lambda i: (0, 0)),
            pl.BlockSpec((1, C), lambda i: (0, 0)),
        ],
        out_specs=[
            pl.BlockSpec((NB, C), lambda i: (i, 0)),
            pl.BlockSpec((3, NB, C), lambda i: (0, i, 0)),
        ],
        out_shape=[
            jax.ShapeDtypeStruct((N, C), jnp.float32),
            jax.ShapeDtypeStruct((3, N, C), jnp.float32),
        ],
    )(rcat, nf0, nf1x, nf1y, nf1z, w0a, w0b, w0c, w1a, w1b, wg, wn, bg2, bn2)


def kernel(node_feat_0, node_feat_1, edge_vec, distances, idx_i, idx_j,
           atomic_number, W_rbf, W0a, W0b, W0c, W1a, W1b, Wg, Wn, bg, bn,
           centers):
    nf1p = jnp.transpose(node_feat_1, (2, 0, 1))
    nf1x, nf1y, nf1z = nf1p[0], nf1p[1], nf1p[2]
    idxj2 = idx_j.astype(jnp.int32).reshape(NW, STEPS, CHUNK)
    idxi2 = idx_i.astype(jnp.int32).reshape(NS, 2 * STEPS, CHUNK)

    tbl = jnp.concatenate([node_feat_0, nf1x, nf1y, nf1z], axis=1)
    g = _gather(tbl, idxj2)
    msgs = _messages(g, edge_vec, distances.reshape(E, 1),
                     centers.reshape(1, NRBF), W_rbf)
    zeros = jnp.zeros((NPAD, C), jnp.float32)
    rcat = _scatter(msgs.reshape(4 * E, C), idxi2,
                    zeros).reshape(4, NPAD, C)[:, :N, :]
    out0, out1p = _node(rcat, node_feat_0, nf1x, nf1y, nf1z,
                        W0a, W0b, W0c, W1a, W1b, Wg, Wn,
                        bg.reshape(1, C), bn.reshape(1, C))
    return out0, jnp.transpose(out1p, (1, 2, 0))


# speedup vs baseline: 22.6283x; 1.2045x over previous
import functools

import jax
import jax.numpy as jnp
from jax import lax
from jax.experimental import pallas as pl
from jax.experimental.pallas import tpu as pltpu
from jax.experimental.pallas import tpu_sc as plsc

N = 10000
E = 320000
C = 128
NRBF = 16
GAMMA = 4.0
INV_NORM = 1.0 / 32.0

NC = 2
NS = 16
NW = NC * NS
EPT = E // NW
CHUNK = 80
STEPS = EPT // CHUNK
NPAD = 10240
RPT = NPAD // NS

EB = 1280
NB = 1000


def _mesh():
    return plsc.VectorSubcoreMesh(core_axis_name="c", subcore_axis_name="s",
                                  num_cores=NC, num_subcores=NS)


def _gather(tbl, idxj2):
    @functools.partial(
        pl.kernel,
        out_type=jax.ShapeDtypeStruct((E, 4 * C), jnp.float32),
        mesh=_mesh(),
        scratch_types=[
            pltpu.VMEM((STEPS, CHUNK), jnp.int32),
            pltpu.VMEM((2, CHUNK, 4 * C), jnp.float32),
            pltpu.SemaphoreType.DMA,
            pltpu.SemaphoreType.DMA,
            pltpu.SemaphoreType.DMA,
            pltpu.SemaphoreType.DMA,
        ],
    )
    def k(tbl_h, idx_h, out_h, idx_v, rows_v, g0, g1, w0, w1):
        c = lax.axis_index("c")
        s = lax.axis_index("s")
        wid = s * NC + c
        pltpu.sync_copy(idx_h.at[wid], idx_v)
        gsem = (g0, g1)
        wsem = (w0, w1)

        def gcp(b, j):
            return pltpu.make_async_copy(tbl_h.at[idx_v.at[j]], rows_v.at[b],
                                         gsem[b])

        def wcp(b, j):
            e0 = wid * EPT + j * CHUNK
            return pltpu.make_async_copy(rows_v.at[b],
                                         out_h.at[pl.ds(e0, CHUNK)], wsem[b])

        gcp(0, 0).start()

        def pair(jj, carry):
            j0 = 2 * jj
            j1 = j0 + 1
            gcp(1, j1).start()
            gcp(0, j0).wait()
            wcp(0, j0).start()
            gcp(1, j1).wait()
            wcp(1, j1).start()
            wcp(0, j0).wait()
            gcp(0, j0 + 2).start()
            wcp(1, j1).wait()
            return carry

        lax.fori_loop(0, (STEPS - 1) // 2, pair, 0, unroll=False)
        jt = STEPS - 1
        gcp(0, jt).wait()
        wcp(0, jt).start()
        wcp(0, jt).wait()

    return k(tbl, idxj2)


def _messages(g, edge_vec, dist2, centers2, w_rbf):
    def body(ev_ref, d_ref, cen_ref, w_ref, g_ref, m_ref):
        ev = ev_ref[...]
        nrm = jnp.sqrt(jnp.sum(ev * ev, axis=1, keepdims=True))
        evn = ev / (nrm + 1e-8)
        d = d_ref[...]
        diff = d - cen_ref[...]
        rbf = jnp.exp(-GAMMA * diff * diff)
        w = w_ref[...]
        f0 = jnp.dot(rbf, w[0], preferred_element_type=jnp.float32)
        f1 = jnp.dot(rbf, w[1], preferred_element_type=jnp.float32)
        f2 = jnp.dot(rbf, w[2], preferred_element_type=jnp.float32)
        f3 = jnp.dot(rbf, w[3], preferred_element_type=jnp.float32)
        gall = g_ref[...]
        g0 = gall[:, 0 * C:1 * C]
        g1x = gall[:, 1 * C:2 * C]
        g1y = gall[:, 2 * C:3 * C]
        g1z = gall[:, 3 * C:4 * C]
        evx = evn[:, 0:1]
        evy = evn[:, 1:2]
        evz = evn[:, 2:3]
        dot1 = g1x * evx + g1y * evy + g1z * evz
        m_ref[0] = f0 * g0 + f3 * dot1
        t = f1 * g0
        m_ref[1] = t * evx + f2 * g1x
        m_ref[2] = t * evy + f2 * g1y
        m_ref[3] = t * evz + f2 * g1z

    return pl.pallas_call(
        body,
        grid=(E // EB,),
        in_specs=[
            pl.BlockSpec((EB, 3), lambda i: (i, 0)),
            pl.BlockSpec((EB, 1), lambda i: (i, 0)),
            pl.BlockSpec((1, NRBF), lambda i: (0, 0)),
            pl.BlockSpec((4, NRBF, C), lambda i: (0, 0, 0)),
            pl.BlockSpec((EB, 4 * C), lambda i: (i, 0)),
        ],
        out_specs=pl.BlockSpec((4, EB, C), lambda i: (0, i, 0)),
        out_shape=jax.ShapeDtypeStruct((4, E, C), jnp.float32),
    )(edge_vec, dist2, centers2, w_rbf, g)


def _scatter(msgs_flat, idxi2, zeros):
    @functools.partial(
        pl.kernel,
        out_type=jax.ShapeDtypeStruct((4 * NPAD, C), jnp.float32),
        mesh=_mesh(),
        scratch_types=[
            pltpu.VMEM((STEPS, CHUNK), jnp.int32),
            pltpu.VMEM((2, CHUNK, C), jnp.float32),
            pltpu.VMEM_SHARED((NPAD, C), jnp.float32),
            pltpu.SemaphoreType.DMA,
            pltpu.SemaphoreType.DMA,
        ],
    )
    def k(m_h, idx_h, z_h, out_h, idx_v, buf_v, acc, l0, l1):
        c = lax.axis_index("c")
        s = lax.axis_index("s")
        row0 = s * RPT
        lsem = (l0, l1)

        for kk in range(2):
            chunk = kk * NC + c

            pltpu.sync_copy(z_h.at[pl.ds(row0, RPT)], acc.at[pl.ds(row0, RPT)])
            plsc.subcore_barrier()

            for h in range(2):
                pltpu.sync_copy(idx_h.at[s, h], idx_v)

                def lcp(b, j):
                    e0 = s * (2 * EPT) + (h * STEPS + j) * CHUNK
                    return pltpu.make_async_copy(
                        m_h.at[pl.ds(chunk * E + e0, CHUNK)], buf_v.at[b],
                        lsem[b])

                def scat(b, j):
                    pltpu.sync_copy(buf_v.at[b], acc.at[idx_v.at[j]],
                                    add=True)

                lcp(0, 0).start()

                def pair(jj, carry):
                    j0 = 2 * jj
                    j1 = j0 + 1
                    lcp(1, j1).start()
                    lcp(0, j0).wait()
                    scat(0, j0)
                    lcp(0, j0 + 2).start()
                    lcp(1, j1).wait()
                    scat(1, j1)
                    return carry

                lax.fori_loop(0, (STEPS - 1) // 2, pair, 0, unroll=False)
                jt = STEPS - 1
                lcp(0, jt).wait()
                scat(0, jt)

            plsc.subcore_barrier()
            pltpu.sync_copy(
                acc.at[pl.ds(row0, RPT)],
                out_h.at[pl.ds(chunk * NPAD + row0, RPT)],
            )

    return k(msgs_flat, idxi2, zeros)


def _node(rcat, nf0, nf1x, nf1y, nf1z, w0a, w0b, w0c, w1a, w1b, wg, wn, bg2, bn2):
    def body(r_ref, n0_ref, x_ref, y_ref, z_ref, a_ref, b_ref, c_ref, d_ref,
             e_ref, g_ref, n_ref, bg_ref, bn_ref, o0_ref, o1_ref):
        r0 = r_ref[0] * INV_NORM
        r1x = r_ref[1] * INV_NORM
        r1y = r_ref[2] * INV_NORM
        r1z = r_ref[3] * INV_NORM
        W0a = a_ref[...]
        W0b = b_ref[...]
        W0c = c_ref[...]
        W1a = d_ref[...]
        W1b = e_ref[...]
        Wg = g_ref[...]
        Wn = n_ref[...]
        y0 = (jnp.dot(r0, W0a, preferred_element_type=jnp.float32)
              + jnp.dot(r0 * r0, W0b, preferred_element_type=jnp.float32)
              + jnp.dot(r1x * r1x + r1y * r1y + r1z * r1z, W0c,
                        preferred_element_type=jnp.float32))
        gate = jax.nn.silu(jnp.dot(y0, Wg, preferred_element_type=jnp.float32)
                           + bg_ref[...])
        u0 = jax.nn.silu(jnp.dot(y0, Wn, preferred_element_type=jnp.float32)
                         + bn_ref[...])
        o0_ref[...] = n0_ref[...] + u0
        y1x = (jnp.dot(r1x, W1a, preferred_element_type=jnp.float32)
               + jnp.dot(r0 * r1x, W1b, preferred_element_type=jnp.float32))
        y1y = (jnp.dot(r1y, W1a, preferred_element_type=jnp.float32)
               + jnp.dot(r0 * r1y, W1b, preferred_element_type=jnp.float32))
        y1z = (jnp.dot(r1z, W1a, preferred_element_type=jnp.float32)
               + jnp.dot(r0 * r1z, W1b, preferred_element_type=jnp.float32))
        o1_ref[0] = x_ref[...] + y1x * gate
        o1_ref[1] = y_ref[...] + y1y * gate
        o1_ref[2] = z_ref[...] + y1z * gate

    return pl.pallas_call(
        body,
        grid=(N // NB,),
        in_specs=[
            pl.BlockSpec((4, NB, C), lambda i: (0, i, 0)),
            pl.BlockSpec((NB, C), lambda i: (i, 0)),
            pl.BlockSpec((NB, C), lambda i: (i, 0)),
            pl.BlockSpec((NB, C), lambda i: (i, 0)),
            pl.BlockSpec((NB, C), lambda i: (i, 0)),
            pl.BlockSpec((C, C), lambda i: (0, 0)),
            pl.BlockSpec((C, C), lambda i: (0, 0)),
            pl.BlockSpec((C, C), lambda i: (0, 0)),
            pl.BlockSpec((C, C), lambda i: (0, 0)),
            pl.BlockSpec((C, C), lambda i: (0, 0)),
            pl.BlockSpec((C, C), lambda i: (0, 0)),
            pl.BlockSpec((C, C), lambda i: (0, 0)),
            pl.BlockSpec((1, C), lambda i: (0, 0)),
            pl.BlockSpec((1, C), lambda i: (0, 0)),
        ],
        out_specs=[
            pl.BlockSpec((NB, C), lambda i: (i, 0)),
            pl.BlockSpec((3, NB, C), lambda i: (0, i, 0)),
        ],
        out_shape=[
            jax.ShapeDtypeStruct((N, C), jnp.float32),
            jax.ShapeDtypeStruct((3, N, C), jnp.float32),
        ],
    )(rcat, nf0, nf1x, nf1y, nf1z, w0a, w0b, w0c, w1a, w1b, wg, wn, bg2, bn2)


def kernel(node_feat_0, node_feat_1, edge_vec, distances, idx_i, idx_j,
           atomic_number, W_rbf, W0a, W0b, W0c, W1a, W1b, Wg, Wn, bg, bn,
           centers):
    nf1p = jnp.transpose(node_feat_1, (2, 0, 1))
    nf1x, nf1y, nf1z = nf1p[0], nf1p[1], nf1p[2]
    idxj2 = idx_j.astype(jnp.int32).reshape(NW, STEPS, CHUNK)
    idxi2 = idx_i.astype(jnp.int32).reshape(NS, 2, STEPS, CHUNK)

    tbl = jnp.concatenate([node_feat_0, nf1x, nf1y, nf1z], axis=1)
    g = _gather(tbl, idxj2)
    msgs = _messages(g, edge_vec, distances.reshape(E, 1),
                     centers.reshape(1, NRBF), W_rbf)
    zeros = jnp.zeros((NPAD, C), jnp.float32)
    rcat = _scatter(msgs.reshape(4 * E, C), idxi2,
                    zeros).reshape(4, NPAD, C)[:, :N, :]
    out0, out1p = _node(rcat, node_feat_0, nf1x, nf1y, nf1z,
                        W0a, W0b, W0c, W1a, W1b, Wg, Wn,
                        bg.reshape(1, C), bn.reshape(1, C))
    return out0, jnp.transpose(out1p, (1, 2, 0))


# speedup vs baseline: 23.2174x; 1.0260x over previous
import functools

import jax
import jax.numpy as jnp
from jax import lax
from jax.experimental import pallas as pl
from jax.experimental.pallas import tpu as pltpu
from jax.experimental.pallas import tpu_sc as plsc

N = 10000
E = 320000
C = 128
NRBF = 16
GAMMA = 4.0
INV_NORM = 1.0 / 32.0

NC = 2
NS = 16
NW = NC * NS
EPT = E // NW
CHUNK = 80
STEPS = EPT // CHUNK
NPAD = 10240
RPT = NPAD // NS

EB = 2560
NB = 1000


def _mesh():
    return plsc.VectorSubcoreMesh(core_axis_name="c", subcore_axis_name="s",
                                  num_cores=NC, num_subcores=NS)


def _gather(tbl, idxj2):
    @functools.partial(
        pl.kernel,
        out_type=jax.ShapeDtypeStruct((E, 4 * C), jnp.float32),
        mesh=_mesh(),
        scratch_types=[
            pltpu.VMEM((STEPS, CHUNK), jnp.int32),
            pltpu.VMEM((2, CHUNK, 4 * C), jnp.float32),
            pltpu.SemaphoreType.DMA,
            pltpu.SemaphoreType.DMA,
            pltpu.SemaphoreType.DMA,
            pltpu.SemaphoreType.DMA,
        ],
    )
    def k(tbl_h, idx_h, out_h, idx_v, rows_v, g0, g1, w0, w1):
        c = lax.axis_index("c")
        s = lax.axis_index("s")
        wid = s * NC + c
        pltpu.sync_copy(idx_h.at[wid], idx_v)
        gsem = (g0, g1)
        wsem = (w0, w1)

        def gcp(b, j):
            return pltpu.make_async_copy(tbl_h.at[idx_v.at[j]], rows_v.at[b],
                                         gsem[b])

        def wcp(b, j):
            e0 = wid * EPT + j * CHUNK
            return pltpu.make_async_copy(rows_v.at[b],
                                         out_h.at[pl.ds(e0, CHUNK)], wsem[b])

        gcp(0, 0).start()

        def pair(jj, carry):
            j0 = 2 * jj
            j1 = j0 + 1
            gcp(1, j1).start()
            gcp(0, j0).wait()
            wcp(0, j0).start()
            gcp(1, j1).wait()
            wcp(1, j1).start()
            wcp(0, j0).wait()
            gcp(0, j0 + 2).start()
            wcp(1, j1).wait()
            return carry

        lax.fori_loop(0, (STEPS - 1) // 2, pair, 0, unroll=False)
        jt = STEPS - 1
        gcp(0, jt).wait()
        wcp(0, jt).start()
        wcp(0, jt).wait()

    return k(tbl, idxj2)


def _messages(g, edge_vec, dist2, centers2, w_rbf):
    def body(ev_ref, d_ref, cen_ref, w_ref, g_ref, m_ref):
        ev = ev_ref[...]
        nrm = jnp.sqrt(jnp.sum(ev * ev, axis=1, keepdims=True))
        evn = ev / (nrm + 1e-8)
        d = d_ref[...]
        diff = d - cen_ref[...]
        rbf = jnp.exp(-GAMMA * diff * diff)
        w = w_ref[...]
        f0 = jnp.dot(rbf, w[0], preferred_element_type=jnp.float32)
        f1 = jnp.dot(rbf, w[1], preferred_element_type=jnp.float32)
        f2 = jnp.dot(rbf, w[2], preferred_element_type=jnp.float32)
        f3 = jnp.dot(rbf, w[3], preferred_element_type=jnp.float32)
        gall = g_ref[...]
        g0 = gall[:, 0 * C:1 * C]
        g1x = gall[:, 1 * C:2 * C]
        g1y = gall[:, 2 * C:3 * C]
        g1z = gall[:, 3 * C:4 * C]
        evx = evn[:, 0:1]
        evy = evn[:, 1:2]
        evz = evn[:, 2:3]
        dot1 = g1x * evx + g1y * evy + g1z * evz
        m_ref[0] = f0 * g0 + f3 * dot1
        t = f1 * g0
        m_ref[1] = t * evx + f2 * g1x
        m_ref[2] = t * evy + f2 * g1y
        m_ref[3] = t * evz + f2 * g1z

    return pl.pallas_call(
        body,
        grid=(E // EB,),
        in_specs=[
            pl.BlockSpec((EB, 3), lambda i: (i, 0)),
            pl.BlockSpec((EB, 1), lambda i: (i, 0)),
            pl.BlockSpec((1, NRBF), lambda i: (0, 0)),
            pl.BlockSpec((4, NRBF, C), lambda i: (0, 0, 0)),
            pl.BlockSpec((EB, 4 * C), lambda i: (i, 0)),
        ],
        out_specs=pl.BlockSpec((4, EB, C), lambda i: (0, i, 0)),
        out_shape=jax.ShapeDtypeStruct((4, E, C), jnp.float32),
    )(edge_vec, dist2, centers2, w_rbf, g)


def _scatter(msgs_flat, idxi2, zeros):
    @functools.partial(
        pl.kernel,
        out_type=jax.ShapeDtypeStruct((4 * NPAD, C), jnp.float32),
        mesh=_mesh(),
        scratch_types=[
            pltpu.VMEM((STEPS, CHUNK), jnp.int32),
            pltpu.VMEM((2, CHUNK, C), jnp.float32),
            pltpu.VMEM_SHARED((NPAD, C), jnp.float32),
            pltpu.SemaphoreType.DMA,
            pltpu.SemaphoreType.DMA,
        ],
    )
    def k(m_h, idx_h, z_h, out_h, idx_v, buf_v, acc, l0, l1):
        c = lax.axis_index("c")
        s = lax.axis_index("s")
        row0 = s * RPT
        lsem = (l0, l1)

        for kk in range(2):
            chunk = kk * NC + c

            pltpu.sync_copy(z_h.at[pl.ds(row0, RPT)], acc.at[pl.ds(row0, RPT)])
            plsc.subcore_barrier()

            for h in range(2):
                pltpu.sync_copy(idx_h.at[s, h], idx_v)

                def lcp(b, j):
                    e0 = s * (2 * EPT) + (h * STEPS + j) * CHUNK
                    return pltpu.make_async_copy(
                        m_h.at[pl.ds(chunk * E + e0, CHUNK)], buf_v.at[b],
                        lsem[b])

                def scat(b, j):
                    pltpu.sync_copy(buf_v.at[b], acc.at[idx_v.at[j]],
                                    add=True)

                lcp(0, 0).start()

                def pair(jj, carry):
                    j0 = 2 * jj
                    j1 = j0 + 1
                    lcp(1, j1).start()
                    lcp(0, j0).wait()
                    scat(0, j0)
                    lcp(0, j0 + 2).start()
                    lcp(1, j1).wait()
                    scat(1, j1)
                    return carry

                lax.fori_loop(0, (STEPS - 1) // 2, pair, 0, unroll=False)
                jt = STEPS - 1
                lcp(0, jt).wait()
                scat(0, jt)

            plsc.subcore_barrier()
            pltpu.sync_copy(
                acc.at[pl.ds(row0, RPT)],
                out_h.at[pl.ds(chunk * NPAD + row0, RPT)],
            )

    return k(msgs_flat, idxi2, zeros)


def _node(rcat, nf0, nf1x, nf1y, nf1z, w0a, w0b, w0c, w1a, w1b, wg, wn, bg2, bn2):
    def body(r_ref, n0_ref, x_ref, y_ref, z_ref, a_ref, b_ref, c_ref, d_ref,
             e_ref, g_ref, n_ref, bg_ref, bn_ref, o0_ref, o1_ref):
        r0 = r_ref[0] * INV_NORM
        r1x = r_ref[1] * INV_NORM
        r1y = r_ref[2] * INV_NORM
        r1z = r_ref[3] * INV_NORM
        W0a = a_ref[...]
        W0b = b_ref[...]
        W0c = c_ref[...]
        W1a = d_ref[...]
        W1b = e_ref[...]
        Wg = g_ref[...]
        Wn = n_ref[...]
        y0 = (jnp.dot(r0, W0a, preferred_element_type=jnp.float32)
              + jnp.dot(r0 * r0, W0b, preferred_element_type=jnp.float32)
              + jnp.dot(r1x * r1x + r1y * r1y + r1z * r1z, W0c,
                        preferred_element_type=jnp.float32))
        gate = jax.nn.silu(jnp.dot(y0, Wg, preferred_element_type=jnp.float32)
                           + bg_ref[...])
        u0 = jax.nn.silu(jnp.dot(y0, Wn, preferred_element_type=jnp.float32)
                         + bn_ref[...])
        o0_ref[...] = n0_ref[...] + u0
        y1x = (jnp.dot(r1x, W1a, preferred_element_type=jnp.float32)
               + jnp.dot(r0 * r1x, W1b, preferred_element_type=jnp.float32))
        y1y = (jnp.dot(r1y, W1a, preferred_element_type=jnp.float32)
               + jnp.dot(r0 * r1y, W1b, preferred_element_type=jnp.float32))
        y1z = (jnp.dot(r1z, W1a, preferred_element_type=jnp.float32)
               + jnp.dot(r0 * r1z, W1b, preferred_element_type=jnp.float32))
        o1_ref[0] = x_ref[...] + y1x * gate
        o1_ref[1] = y_ref[...] + y1y * gate
        o1_ref[2] = z_ref[...] + y1z * gate

    return pl.pallas_call(
        body,
        grid=(N // NB,),
        in_specs=[
            pl.BlockSpec((4, NB, C), lambda i: (0, i, 0)),
            pl.BlockSpec((NB, C), lambda i: (i, 0)),
            pl.BlockSpec((NB, C), lambda i: (i, 0)),
            pl.BlockSpec((NB, C), lambda i: (i, 0)),
            pl.BlockSpec((NB, C), lambda i: (i, 0)),
            pl.BlockSpec((C, C), lambda i: (0, 0)),
            pl.BlockSpec((C, C), lambda i: (0, 0)),
            pl.BlockSpec((C, C), lambda i: (0, 0)),
            pl.BlockSpec((C, C), lambda i: (0, 0)),
            pl.BlockSpec((C, C), lambda i: (0, 0)),
            pl.BlockSpec((C, C), lambda i: (0, 0)),
            pl.BlockSpec((C, C), lambda i: (0, 0)),
            pl.BlockSpec((1, C), lambda i: (0, 0)),
            pl.BlockSpec((1, C), lambda i: (0, 0)),
        ],
        out_specs=[
            pl.BlockSpec((NB, C), lambda i: (i, 0)),
            pl.BlockSpec((3, NB, C), lambda i: (0, i, 0)),
        ],
        out_shape=[
            jax.ShapeDtypeStruct((N, C), jnp.float32),
            jax.ShapeDtypeStruct((3, N, C), jnp.float32),
        ],
    )(rcat, nf0, nf1x, nf1y, nf1z, w0a, w0b, w0c, w1a, w1b, wg, wn, bg2, bn2)


def kernel(node_feat_0, node_feat_1, edge_vec, distances, idx_i, idx_j,
           atomic_number, W_rbf, W0a, W0b, W0c, W1a, W1b, Wg, Wn, bg, bn,
           centers):
    nf1p = jnp.transpose(node_feat_1, (2, 0, 1))
    nf1x, nf1y, nf1z = nf1p[0], nf1p[1], nf1p[2]
    idxj2 = idx_j.astype(jnp.int32).reshape(NW, STEPS, CHUNK)
    idxi2 = idx_i.astype(jnp.int32).reshape(NS, 2, STEPS, CHUNK)

    tbl = jnp.concatenate([node_feat_0, nf1x, nf1y, nf1z], axis=1)
    g = _gather(tbl, idxj2)
    msgs = _messages(g, edge_vec, distances.reshape(E, 1),
                     centers.reshape(1, NRBF), W_rbf)
    zeros = jnp.zeros((NPAD, C), jnp.float32)
    rcat = _scatter(msgs.reshape(4 * E, C), idxi2,
                    zeros).reshape(4, NPAD, C)[:, :N, :]
    out0, out1p = _node(rcat, node_feat_0, nf1x, nf1y, nf1z,
                        W0a, W0b, W0c, W1a, W1b, Wg, Wn,
                        bg.reshape(1, C), bn.reshape(1, C))
    return out0, jnp.transpose(out1p, (1, 2, 0))


# speedup vs baseline: 23.4476x; 1.0099x over previous
import functools

import jax
import jax.numpy as jnp
from jax import lax
from jax.experimental import pallas as pl
from jax.experimental.pallas import tpu as pltpu
from jax.experimental.pallas import tpu_sc as plsc

N = 10000
E = 320000
C = 128
NRBF = 16
GAMMA = 4.0
INV_NORM = 1.0 / 32.0

NC = 2
NS = 16
NW = NC * NS
EPT = E // NW
CHUNK = 80
STEPS = EPT // CHUNK
NPAD = 10240
RPT = NPAD // NS

EB = 3200
NB = 2000


def _mesh():
    return plsc.VectorSubcoreMesh(core_axis_name="c", subcore_axis_name="s",
                                  num_cores=NC, num_subcores=NS)


def _gather(tbl, idxj2):
    @functools.partial(
        pl.kernel,
        out_type=jax.ShapeDtypeStruct((E, 4 * C), jnp.float32),
        mesh=_mesh(),
        scratch_types=[
            pltpu.VMEM((STEPS, CHUNK), jnp.int32),
            pltpu.VMEM((2, CHUNK, 4 * C), jnp.float32),
            pltpu.SemaphoreType.DMA,
            pltpu.SemaphoreType.DMA,
            pltpu.SemaphoreType.DMA,
            pltpu.SemaphoreType.DMA,
        ],
    )
    def k(tbl_h, idx_h, out_h, idx_v, rows_v, g0, g1, w0, w1):
        c = lax.axis_index("c")
        s = lax.axis_index("s")
        wid = s * NC + c
        pltpu.sync_copy(idx_h.at[wid], idx_v)
        gsem = (g0, g1)
        wsem = (w0, w1)

        def gcp(b, j):
            return pltpu.make_async_copy(tbl_h.at[idx_v.at[j]], rows_v.at[b],
                                         gsem[b])

        def wcp(b, j):
            e0 = wid * EPT + j * CHUNK
            return pltpu.make_async_copy(rows_v.at[b],
                                         out_h.at[pl.ds(e0, CHUNK)], wsem[b])

        gcp(0, 0).start()

        def pair(jj, carry):
            j0 = 2 * jj
            j1 = j0 + 1
            gcp(1, j1).start()
            gcp(0, j0).wait()
            wcp(0, j0).start()
            gcp(1, j1).wait()
            wcp(1, j1).start()
            wcp(0, j0).wait()
            gcp(0, j0 + 2).start()
            wcp(1, j1).wait()
            return carry

        lax.fori_loop(0, (STEPS - 1) // 2, pair, 0, unroll=False)
        jt = STEPS - 1
        gcp(0, jt).wait()
        wcp(0, jt).start()
        wcp(0, jt).wait()

    return k(tbl, idxj2)


def _messages(g, edge_vec, dist2, centers2, w_rbf):
    def body(ev_ref, d_ref, cen_ref, w_ref, g_ref, m_ref):
        ev = ev_ref[...]
        nrm = jnp.sqrt(jnp.sum(ev * ev, axis=1, keepdims=True))
        evn = ev / (nrm + 1e-8)
        d = d_ref[...]
        diff = d - cen_ref[...]
        rbf = jnp.exp(-GAMMA * diff * diff)
        w = w_ref[...]
        f0 = jnp.dot(rbf, w[0], preferred_element_type=jnp.float32)
        f1 = jnp.dot(rbf, w[1], preferred_element_type=jnp.float32)
        f2 = jnp.dot(rbf, w[2], preferred_element_type=jnp.float32)
        f3 = jnp.dot(rbf, w[3], preferred_element_type=jnp.float32)
        gall = g_ref[...]
        g0 = gall[:, 0 * C:1 * C]
        g1x = gall[:, 1 * C:2 * C]
        g1y = gall[:, 2 * C:3 * C]
        g1z = gall[:, 3 * C:4 * C]
        evx = evn[:, 0:1]
        evy = evn[:, 1:2]
        evz = evn[:, 2:3]
        dot1 = g1x * evx + g1y * evy + g1z * evz
        m_ref[0] = f0 * g0 + f3 * dot1
        t = f1 * g0
        m_ref[1] = t * evx + f2 * g1x
        m_ref[2] = t * evy + f2 * g1y
        m_ref[3] = t * evz + f2 * g1z

    return pl.pallas_call(
        body,
        grid=(E // EB,),
        in_specs=[
            pl.BlockSpec((EB, 3), lambda i: (i, 0)),
            pl.BlockSpec((EB, 1), lambda i: (i, 0)),
            pl.BlockSpec((1, NRBF), lambda i: (0, 0)),
            pl.BlockSpec((4, NRBF, C), lambda i: (0, 0, 0)),
            pl.BlockSpec((EB, 4 * C), lambda i: (i, 0)),
        ],
        out_specs=pl.BlockSpec((4, EB, C), lambda i: (0, i, 0)),
        out_shape=jax.ShapeDtypeStruct((4, E, C), jnp.float32),
    )(edge_vec, dist2, centers2, w_rbf, g)


def _scatter(msgs_flat, idxi2, zeros):
    @functools.partial(
        pl.kernel,
        out_type=jax.ShapeDtypeStruct((4 * NPAD, C), jnp.float32),
        mesh=_mesh(),
        scratch_types=[
            pltpu.VMEM((STEPS, CHUNK), jnp.int32),
            pltpu.VMEM((2, CHUNK, C), jnp.float32),
            pltpu.VMEM_SHARED((NPAD, C), jnp.float32),
            pltpu.SemaphoreType.DMA,
            pltpu.SemaphoreType.DMA,
        ],
    )
    def k(m_h, idx_h, z_h, out_h, idx_v, buf_v, acc, l0, l1):
        c = lax.axis_index("c")
        s = lax.axis_index("s")
        row0 = s * RPT
        lsem = (l0, l1)

        for kk in range(2):
            chunk = kk * NC + c

            pltpu.sync_copy(z_h.at[pl.ds(row0, RPT)], acc.at[pl.ds(row0, RPT)])
            plsc.subcore_barrier()

            for h in range(2):
                pltpu.sync_copy(idx_h.at[s, h], idx_v)

                def lcp(b, j):
                    e0 = s * (2 * EPT) + (h * STEPS + j) * CHUNK
                    return pltpu.make_async_copy(
                        m_h.at[pl.ds(chunk * E + e0, CHUNK)], buf_v.at[b],
                        lsem[b])

                def scat(b, j):
                    pltpu.sync_copy(buf_v.at[b], acc.at[idx_v.at[j]],
                                    add=True)

                lcp(0, 0).start()

                def pair(jj, carry):
                    j0 = 2 * jj
                    j1 = j0 + 1
                    lcp(1, j1).start()
                    lcp(0, j0).wait()
                    scat(0, j0)
                    lcp(0, j0 + 2).start()
                    lcp(1, j1).wait()
                    scat(1, j1)
                    return carry

                lax.fori_loop(0, (STEPS - 1) // 2, pair, 0, unroll=False)
                jt = STEPS - 1
                lcp(0, jt).wait()
                scat(0, jt)

            plsc.subcore_barrier()
            pltpu.sync_copy(
                acc.at[pl.ds(row0, RPT)],
                out_h.at[pl.ds(chunk * NPAD + row0, RPT)],
            )

    return k(msgs_flat, idxi2, zeros)


def _node(rcat, nf0, nf1x, nf1y, nf1z, w0a, w0b, w0c, w1a, w1b, wg, wn, bg2, bn2):
    def body(r_ref, n0_ref, x_ref, y_ref, z_ref, a_ref, b_ref, c_ref, d_ref,
             e_ref, g_ref, n_ref, bg_ref, bn_ref, o0_ref, o1_ref):
        r0 = r_ref[0] * INV_NORM
        r1x = r_ref[1] * INV_NORM
        r1y = r_ref[2] * INV_NORM
        r1z = r_ref[3] * INV_NORM
        W0a = a_ref[...]
        W0b = b_ref[...]
        W0c = c_ref[...]
        W1a = d_ref[...]
        W1b = e_ref[...]
        Wg = g_ref[...]
        Wn = n_ref[...]
        y0 = (jnp.dot(r0, W0a, preferred_element_type=jnp.float32)
              + jnp.dot(r0 * r0, W0b, preferred_element_type=jnp.float32)
              + jnp.dot(r1x * r1x + r1y * r1y + r1z * r1z, W0c,
                        preferred_element_type=jnp.float32))
        gate = jax.nn.silu(jnp.dot(y0, Wg, preferred_element_type=jnp.float32)
                           + bg_ref[...])
        u0 = jax.nn.silu(jnp.dot(y0, Wn, preferred_element_type=jnp.float32)
                         + bn_ref[...])
        o0_ref[...] = n0_ref[...] + u0
        y1x = (jnp.dot(r1x, W1a, preferred_element_type=jnp.float32)
               + jnp.dot(r0 * r1x, W1b, preferred_element_type=jnp.float32))
        y1y = (jnp.dot(r1y, W1a, preferred_element_type=jnp.float32)
               + jnp.dot(r0 * r1y, W1b, preferred_element_type=jnp.float32))
        y1z = (jnp.dot(r1z, W1a, preferred_element_type=jnp.float32)
               + jnp.dot(r0 * r1z, W1b, preferred_element_type=jnp.float32))
        o1_ref[0] = x_ref[...] + y1x * gate
        o1_ref[1] = y_ref[...] + y1y * gate
        o1_ref[2] = z_ref[...] + y1z * gate

    return pl.pallas_call(
        body,
        grid=(N // NB,),
        in_specs=[
            pl.BlockSpec((4, NB, C), lambda i: (0, i, 0)),
            pl.BlockSpec((NB, C), lambda i: (i, 0)),
            pl.BlockSpec((NB, C), lambda i: (i, 0)),
            pl.BlockSpec((NB, C), lambda i: (i, 0)),
            pl.BlockSpec((NB, C), lambda i: (i, 0)),
            pl.BlockSpec((C, C), lambda i: (0, 0)),
            pl.BlockSpec((C, C), lambda i: (0, 0)),
            pl.BlockSpec((C, C), lambda i: (0, 0)),
            pl.BlockSpec((C, C), lambda i: (0, 0)),
            pl.BlockSpec((C, C), lambda i: (0, 0)),
            pl.BlockSpec((C, C), lambda i: (0, 0)),
            pl.BlockSpec((C, C), lambda i: (0, 0)),
            pl.BlockSpec((1, C), lambda i: (0, 0)),
            pl.BlockSpec((1, C), lambda i: (0, 0)),
        ],
        out_specs=[
            pl.BlockSpec((NB, C), lambda i: (i, 0)),
            pl.BlockSpec((3, NB, C), lambda i: (0, i, 0)),
        ],
        out_shape=[
            jax.ShapeDtypeStruct((N, C), jnp.float32),
            jax.ShapeDtypeStruct((3, N, C), jnp.float32),
        ],
    )(rcat, nf0, nf1x, nf1y, nf1z, w0a, w0b, w0c, w1a, w1b, wg, wn, bg2, bn2)


def kernel(node_feat_0, node_feat_1, edge_vec, distances, idx_i, idx_j,
           atomic_number, W_rbf, W0a, W0b, W0c, W1a, W1b, Wg, Wn, bg, bn,
           centers):
    nf1p = jnp.transpose(node_feat_1, (2, 0, 1))
    nf1x, nf1y, nf1z = nf1p[0], nf1p[1], nf1p[2]
    idxj2 = idx_j.astype(jnp.int32).reshape(NW, STEPS, CHUNK)
    idxi2 = idx_i.astype(jnp.int32).reshape(NS, 2, STEPS, CHUNK)

    tbl = jnp.concatenate([node_feat_0, nf1x, nf1y, nf1z], axis=1)
    g = _gather(tbl, idxj2)
    msgs = _messages(g, edge_vec, distances.reshape(E, 1),
                     centers.reshape(1, NRBF), W_rbf)
    zeros = jnp.zeros((NPAD, C), jnp.float32)
    rcat = _scatter(msgs.reshape(4 * E, C), idxi2,
                    zeros).reshape(4, NPAD, C)[:, :N, :]
    out0, out1p = _node(rcat, node_feat_0, nf1x, nf1y, nf1z,
                        W0a, W0b, W0c, W1a, W1b, Wg, Wn,
                        bg.reshape(1, C), bn.reshape(1, C))
    return out0, jnp.transpose(out1p, (1, 2, 0))
